# Initial kernel scaffold; baseline (speedup 1.0000x reference)
import jax
import jax.numpy as jnp
from jax.experimental import pallas as pl


def kernel(x, table):
    raise NotImplementedError("write your pallas kernel here")


# speedup vs baseline: 1.0235x; 1.0235x over previous
import functools

import jax
import jax.numpy as jnp
from jax import lax
from jax.experimental import pallas as pl
from jax.experimental.pallas import tpu as pltpu
from jax.experimental.pallas import tpu_sc as plsc

VOCAB = 1000000
EMBED_DIM = 32
BATCH = 16384
HIST = 50

_info = plsc.get_sparse_core_info()
NC, NS, L = _info.num_cores, _info.num_subcores, _info.num_lanes
NW = NC * NS

B_TOTAL = BATCH * HIST
B_PER_W = B_TOTAL // NW
CHUNK = 128
N_CHUNKS = B_PER_W // CHUNK

assert B_TOTAL % NW == 0 and B_PER_W % CHUNK == 0


def _make_kernel():
    mesh = plsc.VectorSubcoreMesh(core_axis_name="c", subcore_axis_name="s")

    @functools.partial(
        pl.kernel,
        mesh=mesh,
        out_type=jax.ShapeDtypeStruct((B_TOTAL, EMBED_DIM), jnp.float32),
        scratch_types=[
            pltpu.VMEM((N_CHUNKS, CHUNK), jnp.int32),
            pltpu.VMEM((CHUNK, EMBED_DIM), jnp.float32),
            pltpu.SemaphoreType.DMA,
        ],
        compiler_params=pltpu.CompilerParams(use_tc_tiling_on_sc=False),
    )
    def emb_kernel(x_hbm, table_hbm, out_hbm, idx_v, rows_v, sem):
        wid = lax.axis_index("s") * NC + lax.axis_index("c")
        base = wid * B_PER_W
        pltpu.sync_copy(x_hbm.at[wid], idx_v)

        def step(j, carry):
            pltpu.async_copy(table_hbm.at[idx_v.at[j]], rows_v, sem).wait()
            pltpu.sync_copy(rows_v, out_hbm.at[pl.ds(base + j * CHUNK, CHUNK)])
            return carry

        lax.fori_loop(0, N_CHUNKS, step, 0)

    return emb_kernel


_emb = _make_kernel()


def kernel(x, table):
    x3 = x.reshape(NW, N_CHUNKS, CHUNK).astype(jnp.int32)
    out = _emb(x3, table)
    return out.reshape(BATCH, HIST, EMBED_DIM)


# speedup vs baseline: 1.1113x; 1.0857x over previous
import functools

import jax
import jax.numpy as jnp
from jax import lax
from jax.experimental import pallas as pl
from jax.experimental.pallas import tpu as pltpu
from jax.experimental.pallas import tpu_sc as plsc

VOCAB = 1000000
EMBED_DIM = 32
BATCH = 16384
HIST = 50

_info = plsc.get_sparse_core_info()
NC, NS, L = _info.num_cores, _info.num_subcores, _info.num_lanes
NW = NC * NS

B_TOTAL = BATCH * HIST
B_PER_W = B_TOTAL // NW
CHUNK = 128
N_CHUNKS = B_PER_W // CHUNK
NBUF = 20

assert B_TOTAL % NW == 0 and B_PER_W % CHUNK == 0 and N_CHUNKS % NBUF == 0


def _make_kernel():
    mesh = plsc.VectorSubcoreMesh(core_axis_name="c", subcore_axis_name="s")

    @functools.partial(
        pl.kernel,
        mesh=mesh,
        out_type=jax.ShapeDtypeStruct((B_TOTAL, EMBED_DIM), jnp.float32),
        scratch_types=[
            pltpu.VMEM((N_CHUNKS, CHUNK), jnp.int32),
            pltpu.VMEM((NBUF, CHUNK, EMBED_DIM), jnp.float32),
            pltpu.SemaphoreType.DMA,
            pltpu.SemaphoreType.DMA,
        ],
        compiler_params=pltpu.CompilerParams(use_tc_tiling_on_sc=False),
    )
    def emb_kernel(x_hbm, table_hbm, out_hbm, idx_v, rows_v, gsem, ssem):
        wid = lax.axis_index("s") * NC + lax.axis_index("c")
        base = wid * B_PER_W
        pltpu.sync_copy(x_hbm.at[wid], idx_v)

        def group(g, carry):
            gd = []
            for b in range(NBUF):
                j = g * NBUF + b
                gd.append(pltpu.async_copy(
                    table_hbm.at[idx_v.at[j]], rows_v.at[b], gsem))
            sd = []
            for b in range(NBUF):
                j = g * NBUF + b
                gd[b].wait()
                sd.append(pltpu.async_copy(
                    rows_v.at[b],
                    out_hbm.at[pl.ds(base + j * CHUNK, CHUNK)], ssem))
            for d in sd:
                d.wait()
            return carry

        lax.fori_loop(0, N_CHUNKS // NBUF, group, 0)

    return emb_kernel


_emb = _make_kernel()


def kernel(x, table):
    x3 = x.reshape(NW, N_CHUNKS, CHUNK).astype(jnp.int32)
    out = _emb(x3, table)
    return out.reshape(BATCH, HIST, EMBED_DIM)


# speedup vs baseline: 1.3127x; 1.1813x over previous
import functools

import jax
import jax.numpy as jnp
from jax import lax
from jax.experimental import pallas as pl
from jax.experimental.pallas import tpu as pltpu
from jax.experimental.pallas import tpu_sc as plsc

VOCAB = 1000000
EMBED_DIM = 32
BATCH = 16384
HIST = 50

_info = plsc.get_sparse_core_info()
NC, NS, L = _info.num_cores, _info.num_subcores, _info.num_lanes
NW = NC * NS

B_PER_W = BATCH // NW
CHUNK = 128
S_PER_W = B_PER_W // CHUNK
EO = EMBED_DIM // 8
BT = BATCH // CHUNK

assert BATCH % (NW * CHUNK) == 0


def _make_kernel():
    mesh = plsc.VectorSubcoreMesh(core_axis_name="c", subcore_axis_name="s")

    @functools.partial(
        pl.kernel,
        mesh=mesh,
        out_type=jax.ShapeDtypeStruct((HIST, EO, BT, 8, CHUNK), jnp.float32),
        scratch_types=[
            pltpu.VMEM((CHUNK,), jnp.int32),
            pltpu.VMEM((CHUNK, EMBED_DIM), jnp.float32),
            pltpu.VMEM((1, EO, 1, 8, CHUNK), jnp.float32),
            pltpu.SemaphoreType.DMA,
            pltpu.SemaphoreType.DMA,
        ],
        compiler_params=pltpu.CompilerParams(
            use_tc_tiling_on_sc=False, needs_layout_passes=False),
    )
    def emb_kernel(xt_hbm, table_hbm, out_hbm, idx_v, rows_v, rt_v, gsem, ssem):
        wid = lax.axis_index("s") * NC + lax.axis_index("c")
        b0 = wid * B_PER_W

        def chunk_body(t, carry):
            h = t // S_PER_W
            s = t % S_PER_W
            bt = wid * S_PER_W + s
            pltpu.sync_copy(xt_hbm.at[h, pl.ds(b0 + s * CHUNK, CHUNK)], idx_v)
            pltpu.async_copy(table_hbm.at[idx_v], rows_v, gsem).wait()
            lanes = lax.iota(jnp.int32, L)
            for e in range(EMBED_DIM):
                for k in range(CHUNK // L):
                    v = plsc.load_gather(
                        rows_v, [k * L + lanes, jnp.full((L,), e, jnp.int32)])
                    rt_v[0, e // 8, 0, e % 8, pl.ds(k * L, L)] = v
            pltpu.async_copy(
                rt_v, out_hbm.at[pl.ds(h, 1), :, pl.ds(bt, 1)], ssem).wait()
            return carry

        lax.fori_loop(0, HIST * S_PER_W, chunk_body, 0)

    return emb_kernel


_emb = _make_kernel()


def kernel(x, table):
    xt = x.T.astype(jnp.int32)
    out5 = _emb(xt, table)
    t = out5.transpose((2, 4, 0, 1, 3))
    return t.reshape(BATCH, HIST, EMBED_DIM)


# speedup vs baseline: 1.4764x; 1.1247x over previous
import functools

import jax
import jax.numpy as jnp
from jax import lax
from jax.experimental import pallas as pl
from jax.experimental.pallas import tpu as pltpu
from jax.experimental.pallas import tpu_sc as plsc

VOCAB = 1000000
EMBED_DIM = 32
BATCH = 16384
HIST = 50

_info = plsc.get_sparse_core_info()
NC, NS, L = _info.num_cores, _info.num_subcores, _info.num_lanes
NW = NC * NS

B_PER_W = BATCH // NW
CHUNK = 128
S_PER_W = B_PER_W // CHUNK
EO = EMBED_DIM // 8
BT = BATCH // CHUNK

assert BATCH % (NW * CHUNK) == 0


NBUF = 4
N_CHUNKS = HIST * S_PER_W
NGROUPS = N_CHUNKS // NBUF

assert N_CHUNKS % NBUF == 0


def _make_kernel():
    mesh = plsc.VectorSubcoreMesh(core_axis_name="c", subcore_axis_name="s")

    @functools.partial(
        pl.kernel,
        mesh=mesh,
        out_type=jax.ShapeDtypeStruct((HIST, EO, BT, 8, CHUNK), jnp.float32),
        scratch_types=(
            [pltpu.VMEM((HIST, S_PER_W, CHUNK), jnp.int32),
             pltpu.VMEM((NBUF, CHUNK, EMBED_DIM), jnp.float32),
             pltpu.VMEM((NBUF, 1, EO, 1, 8, CHUNK), jnp.float32),
             pltpu.SemaphoreType.DMA]
            + [pltpu.SemaphoreType.DMA] * (2 * NBUF)
        ),
        compiler_params=pltpu.CompilerParams(
            use_tc_tiling_on_sc=False, needs_layout_passes=False),
    )
    def emb_kernel(xt_hbm, table_hbm, out_hbm, idx_v, rows_v, rt_v, isem,
                   *sems):
        gsem, ssem = sems[:NBUF], sems[NBUF:]
        wid = lax.axis_index("s") * NC + lax.axis_index("c")

        pltpu.async_copy(
            xt_hbm.at[:, pl.ds(wid * S_PER_W, S_PER_W)], idx_v, isem).wait()

        def fire_gather(j, b):
            h = j // S_PER_W
            s = j % S_PER_W
            pltpu.async_copy(
                table_hbm.at[idx_v.at[h, s]], rows_v.at[b], gsem[b])

        def fire_store(j, b):
            h = j // S_PER_W
            s = j % S_PER_W
            pltpu.async_copy(
                rt_v.at[b],
                out_hbm.at[pl.ds(h, 1), :, pl.ds(wid * S_PER_W + s, 1)],
                ssem[b])

        def wait_gather(b):
            pltpu.make_async_copy(
                table_hbm.at[pl.ds(0, CHUNK)], rows_v.at[b], gsem[b]).wait()

        def wait_store(b):
            pltpu.make_async_copy(
                out_hbm.at[pl.ds(0, 1), :, pl.ds(0, 1)], rt_v.at[b],
                ssem[b]).wait()

        lanes = lax.iota(jnp.int32, L)

        def transpose(b):
            for e in range(EMBED_DIM):
                for k in range(CHUNK // L):
                    v = plsc.load_gather(
                        rows_v.at[b],
                        [k * L + lanes, jnp.full((L,), e, jnp.int32)])
                    rt_v[b, 0, e // 8, 0, e % 8, pl.ds(k * L, L)] = v

        for b in range(NBUF):
            fire_gather(b, b)

        def group(g, carry):
            for b in range(NBUF):
                j = g * NBUF + b
                wait_gather(b)

                @pl.when(g > 0)
                def _():
                    wait_store(b)

                transpose(b)
                fire_store(j, b)

                @pl.when(g < NGROUPS - 1)
                def _():
                    fire_gather(j + NBUF, b)

            return carry

        lax.fori_loop(0, NGROUPS, group, 0)
        for b in range(NBUF):
            wait_store(b)

    return emb_kernel


_emb = _make_kernel()


def kernel(x, table):
    xt = x.T.astype(jnp.int32).reshape(HIST, BT, CHUNK)
    out5 = _emb(xt, table)
    t = out5.transpose((2, 4, 0, 1, 3))
    return t.reshape(BATCH, HIST, EMBED_DIM)


# speedup vs baseline: 2.4091x; 1.6317x over previous
import functools

import jax
import jax.numpy as jnp
from jax import lax
from jax.experimental import pallas as pl
from jax.experimental.pallas import tpu as pltpu
from jax.experimental.pallas import tpu_sc as plsc

VOCAB = 1000000
EMBED_DIM = 32
BATCH = 16384
HIST = 50

_info = plsc.get_sparse_core_info()
NC, NS, L = _info.num_cores, _info.num_subcores, _info.num_lanes
NW = NC * NS

B_PER_W = BATCH // NW
CHUNK = 128
S_PER_W = B_PER_W // CHUNK
EO = EMBED_DIM // 8
BT = BATCH // CHUNK

assert BATCH % (NW * CHUNK) == 0


NBUF = 4
N_CHUNKS = HIST * S_PER_W
NGROUPS = N_CHUNKS // NBUF

assert N_CHUNKS % NBUF == 0


def _make_kernel():
    mesh = plsc.VectorSubcoreMesh(core_axis_name="c", subcore_axis_name="s")

    @functools.partial(
        pl.kernel,
        mesh=mesh,
        out_type=jax.ShapeDtypeStruct((HIST, EO, BT, 8, CHUNK), jnp.float32),
        scratch_types=(
            [pltpu.VMEM((HIST, S_PER_W, CHUNK), jnp.int32),
             pltpu.VMEM((NBUF, CHUNK, EMBED_DIM), jnp.float32),
             pltpu.VMEM((NBUF, 1, EO, 1, 8, CHUNK + 1), jnp.float32),
             pltpu.SemaphoreType.DMA]
            + [pltpu.SemaphoreType.DMA] * (2 * NBUF)
        ),
        compiler_params=pltpu.CompilerParams(
            use_tc_tiling_on_sc=False, needs_layout_passes=False),
    )
    def emb_kernel(xt_hbm, table_hbm, out_hbm, idx_v, rows_v, rt_v, isem,
                   *sems):
        gsem, ssem = sems[:NBUF], sems[NBUF:]
        wid = lax.axis_index("s") * NC + lax.axis_index("c")

        pltpu.async_copy(
            xt_hbm.at[:, pl.ds(wid * S_PER_W, S_PER_W)], idx_v, isem).wait()

        def fire_gather(j, b):
            h = j // S_PER_W
            s = j % S_PER_W
            pltpu.async_copy(
                table_hbm.at[idx_v.at[h, s]], rows_v.at[b], gsem[b])

        def rt_view(b):
            return rt_v.at[b, :, :, :, :, pl.ds(0, CHUNK)]

        def fire_store(j, b):
            h = j // S_PER_W
            s = j % S_PER_W
            pltpu.async_copy(
                rt_view(b),
                out_hbm.at[pl.ds(h, 1), :, pl.ds(wid * S_PER_W + s, 1)],
                ssem[b])

        def wait_gather(b):
            pltpu.make_async_copy(
                table_hbm.at[pl.ds(0, CHUNK)], rows_v.at[b], gsem[b]).wait()

        def wait_store(b):
            pltpu.make_async_copy(
                out_hbm.at[pl.ds(0, 1), :, pl.ds(0, 1)], rt_view(b),
                ssem[b]).wait()

        lanes = lax.iota(jnp.int32, L)
        zeros = jnp.zeros((L,), jnp.int32)
        eo_idx = [(q * L + lanes) // 8 for q in range(2)]
        ei_idx = [(q * L + lanes) % 8 for q in range(2)]

        def transpose(b):
            for r in range(CHUNK):
                for q in range(2):
                    v = rows_v[b, r, pl.ds(q * L, L)]
                    plsc.store_scatter(
                        rt_v.at[b],
                        [zeros, eo_idx[q], zeros, ei_idx[q],
                         jnp.full((L,), r, jnp.int32)],
                        v)

        for b in range(NBUF):
            fire_gather(b, b)

        def group(g, carry):
            for b in range(NBUF):
                j = g * NBUF + b
                wait_gather(b)

                @pl.when(g > 0)
                def _():
                    wait_store(b)

                transpose(b)
                fire_store(j, b)

                @pl.when(g < NGROUPS - 1)
                def _():
                    fire_gather(j + NBUF, b)

            return carry

        lax.fori_loop(0, NGROUPS, group, 0)
        for b in range(NBUF):
            wait_store(b)

    return emb_kernel


_emb = _make_kernel()


def kernel(x, table):
    xt = x.T.astype(jnp.int32).reshape(HIST, BT, CHUNK)
    out5 = _emb(xt, table)
    t = out5.transpose((2, 4, 0, 1, 3))
    return t.reshape(BATCH, HIST, EMBED_DIM)


# speedup vs baseline: 2.4541x; 1.0187x over previous
import functools

import jax
import jax.numpy as jnp
from jax import lax
from jax.experimental import pallas as pl
from jax.experimental.pallas import tpu as pltpu
from jax.experimental.pallas import tpu_sc as plsc

VOCAB = 1000000
EMBED_DIM = 32
BATCH = 16384
HIST = 50

_info = plsc.get_sparse_core_info()
NC, NS, L = _info.num_cores, _info.num_subcores, _info.num_lanes
NW = NC * NS

B_PER_W = BATCH // NW
CHUNK = 128
S_PER_W = B_PER_W // CHUNK
EO = EMBED_DIM // 8
BT = BATCH // CHUNK

assert BATCH % (NW * CHUNK) == 0


NBUF = 4
N_CHUNKS = HIST * S_PER_W
NGROUPS = N_CHUNKS // NBUF

assert N_CHUNKS % NBUF == 0


def _make_kernel():
    mesh = plsc.VectorSubcoreMesh(core_axis_name="c", subcore_axis_name="s")

    @functools.partial(
        pl.kernel,
        mesh=mesh,
        out_type=jax.ShapeDtypeStruct((HIST, EO, BT, 8, CHUNK), jnp.float32),
        scratch_types=(
            [pltpu.VMEM((HIST, S_PER_W, CHUNK), jnp.int32),
             pltpu.VMEM((NBUF, CHUNK, EMBED_DIM), jnp.float32),
             pltpu.VMEM((NBUF, 1, EO, 1, 8, CHUNK + 1), jnp.float32),
             pltpu.SemaphoreType.DMA]
            + [pltpu.SemaphoreType.DMA] * (2 * NBUF)
        ),
        compiler_params=pltpu.CompilerParams(
            use_tc_tiling_on_sc=False, needs_layout_passes=False),
    )
    def emb_kernel(xt_hbm, table_hbm, out_hbm, idx_v, rows_v, rt_v, isem,
                   *sems):
        gsem, ssem = sems[:NBUF], sems[NBUF:]
        wid = lax.axis_index("s") * NC + lax.axis_index("c")

        pltpu.async_copy(
            xt_hbm.at[:, pl.ds(wid * S_PER_W, S_PER_W)], idx_v, isem).wait()

        def fire_gather(j, b):
            h = j // S_PER_W
            s = j % S_PER_W
            pltpu.async_copy(
                table_hbm.at[idx_v.at[h, s]], rows_v.at[b], gsem[b])

        def rt_view(b):
            return rt_v.at[b, :, :, :, :, pl.ds(0, CHUNK)]

        def fire_store(j, b):
            h = j // S_PER_W
            s = j % S_PER_W
            pltpu.async_copy(
                rt_view(b),
                out_hbm.at[pl.ds(h, 1), :, pl.ds(wid * S_PER_W + s, 1)],
                ssem[b])

        def wait_gather(b):
            pltpu.make_async_copy(
                table_hbm.at[pl.ds(0, CHUNK)], rows_v.at[b], gsem[b]).wait()

        def wait_store(b):
            pltpu.make_async_copy(
                out_hbm.at[pl.ds(0, 1), :, pl.ds(0, 1)], rt_view(b),
                ssem[b]).wait()

        lanes = lax.iota(jnp.int32, L)
        zeros = jnp.zeros((L,), jnp.int32)
        eo_idx = [(q * L + lanes) // 8 for q in range(2)]
        ei_idx = [(q * L + lanes) % 8 for q in range(2)]

        def transpose(b):
            for r in range(CHUNK):
                for q in range(2):
                    v = rows_v[b, r, pl.ds(q * L, L)]
                    plsc.store_scatter(
                        rt_v.at[b],
                        [zeros, eo_idx[q], zeros, ei_idx[q],
                         jnp.full((L,), r, jnp.int32)],
                        v)

        for b in range(NBUF):
            fire_gather(b, b)

        def group(g, carry):
            for b in range(NBUF):
                j = g * NBUF + b
                wait_gather(b)

                @pl.when(g > 0)
                def _():
                    wait_store(b)

                transpose(b)
                fire_store(j, b)

                @pl.when(g < NGROUPS - 1)
                def _():
                    fire_gather(j + NBUF, b)

            return carry

        lax.fori_loop(0, NGROUPS, group, 0)
        for b in range(NBUF):
            wait_store(b)

    return emb_kernel


_emb = _make_kernel()


def kernel(x, table):
    xt = (x.T.astype(jnp.int32) * 4).reshape(HIST, BT, CHUNK)
    t4 = jnp.pad(table, ((0, 0), (0, 128 - EMBED_DIM))).reshape(-1, EMBED_DIM)
    out5 = _emb(xt, t4)
    t = out5.transpose((2, 4, 0, 1, 3))
    return t.reshape(BATCH, HIST, EMBED_DIM)
